# baseline (speedup 1.0000x reference)
import functools

import jax
import jax.numpy as jnp
from jax import lax
from jax.experimental import pallas as pl
from jax.experimental.pallas import tpu as pltpu
from jax.experimental.pallas import tpu_sc as plsc

_NC, _NS = 2, 16
_NW = _NC * _NS
_B_SC = 6
_CB = 64

_GDN = lax.GatherDimensionNumbers(
    offset_dims=(), collapsed_slice_dims=(0,), start_index_map=(0,))


def _gat(v, idx):
    return lax.gather(v, idx[:, None], _GDN, (1,),
                      mode=lax.GatherScatterMode.PROMISE_IN_BOUNDS)


def _sc_body(x_hbm, idx_hbm, out_hbm, idx_v, in_v, out_v):
    n_b, n_c = x_hbm.shape[0], x_hbm.shape[1]
    spw = (n_b * n_c) // _NW
    wid = lax.axis_index("s") * _NC + lax.axis_index("c")
    pltpu.sync_copy(idx_hbm, idx_v)
    cols = [idx_v[j, :] for j in range(4)]
    mask0 = cols[0] < 16
    mask1 = cols[1] < 16
    a_lo = jnp.where(mask0, cols[0], 0)
    a_hi = jnp.where(mask0, 0, cols[0] - 16)
    b_lo = jnp.where(mask1, cols[1], 0)
    b_hi = jnp.where(mask1, 0, cols[1] - 16)

    def do_slice(i, carry):
        t = wid * spw + i
        b = t // n_c
        c = t % n_c
        pltpu.sync_copy(x_hbm.at[b, c], in_v)

        def row_body(r, cc):
            v0 = in_v[r, 0:16]
            v1 = in_v[r, 16:32]
            v2 = in_v[r, 32:48]
            v3 = in_v[r, 48:64]
            u0 = jnp.maximum(v0, v2)
            u1 = jnp.maximum(v1, v3)
            s0 = jnp.where(mask0, _gat(u0, a_lo), _gat(u1, a_hi))
            s1 = jnp.where(mask1, _gat(u0, b_lo), _gat(u1, b_hi))
            out_v[r, :] = jnp.maximum(s0, s1)
            return cc

        lax.fori_loop(0, 196, row_body, 0, unroll=4)
        pltpu.sync_copy(out_v, out_hbm.at[b, c])
        return carry

    lax.fori_loop(0, spw, do_slice, 0)


def _tc_body(x_ref, s_ref, o_ref):
    xb = x_ref[0]
    sel = s_ref[...]
    dn = (((2,), (0,)), ((), ()))
    t0 = lax.dot_general(xb, sel[0], dn, preferred_element_type=jnp.float32)
    t1 = lax.dot_general(xb, sel[1], dn, preferred_element_type=jnp.float32)
    t2 = lax.dot_general(xb, sel[2], dn, preferred_element_type=jnp.float32)
    t3 = lax.dot_general(xb, sel[3], dn, preferred_element_type=jnp.float32)
    o_ref[0] = jnp.maximum(jnp.maximum(t0, t1), jnp.maximum(t2, t3))


def kernel(x, indices):
    b, c, s, g = x.shape
    p = indices.shape[1]
    idx32 = indices.astype(jnp.int32)

    mesh = plsc.VectorSubcoreMesh(core_axis_name="c", subcore_axis_name="s")
    sc_run = functools.partial(
        pl.kernel,
        out_type=jax.ShapeDtypeStruct((_B_SC, c, s, p), x.dtype),
        mesh=mesh,
        scratch_types=[
            pltpu.VMEM((4, 16), jnp.int32),
            pltpu.VMEM((s, g), jnp.float32),
            pltpu.VMEM((s, p), jnp.float32),
        ],
        compiler_params=pltpu.CompilerParams(needs_layout_passes=False),
    )(_sc_body)
    out_sc = sc_run(x[:_B_SC], idx32)

    sel = (idx32[:, None, :] == jnp.arange(g, dtype=jnp.int32)[None, :, None]
           ).astype(x.dtype)
    b_tc = b - _B_SC
    out_tc = pl.pallas_call(
        _tc_body,
        grid=(b_tc, c // _CB),
        in_specs=[
            pl.BlockSpec((1, _CB, s, g), lambda i, j: (i + _B_SC, j, 0, 0)),
            pl.BlockSpec((4, g, p), lambda i, j: (0, 0, 0)),
        ],
        out_specs=pl.BlockSpec((1, _CB, s, p), lambda i, j: (i, j, 0, 0)),
        out_shape=jax.ShapeDtypeStruct((b_tc, c, s, p), x.dtype),
    )(x, sel)

    return jnp.concatenate([out_sc, out_tc], axis=0)


# speedup vs baseline: 1.1616x; 1.1616x over previous
import jax
import jax.numpy as jnp
from jax import lax
from jax.experimental import pallas as pl


_CB = 64


def _pool_body(x_ref, s_ref, o_ref):
    xb = x_ref[0]
    sel = s_ref[...]
    dn = (((2,), (0,)), ((), ()))
    t0 = lax.dot_general(xb, sel[0], dn, preferred_element_type=jnp.float32)
    t1 = lax.dot_general(xb, sel[1], dn, preferred_element_type=jnp.float32)
    t2 = lax.dot_general(xb, sel[2], dn, preferred_element_type=jnp.float32)
    t3 = lax.dot_general(xb, sel[3], dn, preferred_element_type=jnp.float32)
    o_ref[0] = jnp.maximum(jnp.maximum(t0, t1), jnp.maximum(t2, t3))


def kernel(x, indices):
    b, c, s, g = x.shape
    p = indices.shape[1]
    sel = (indices[:, None, :] == jnp.arange(g, dtype=indices.dtype)[None, :, None]
           ).astype(x.dtype)
    return pl.pallas_call(
        _pool_body,
        grid=(b, c // _CB),
        in_specs=[
            pl.BlockSpec((1, _CB, s, g), lambda i, j: (i, j, 0, 0)),
            pl.BlockSpec((4, g, p), lambda i, j: (0, 0, 0)),
        ],
        out_specs=pl.BlockSpec((1, _CB, s, p), lambda i, j: (i, j, 0, 0)),
        out_shape=jax.ShapeDtypeStruct((b, c, s, p), x.dtype),
    )(x, sel)


# speedup vs baseline: 1.1621x; 1.0004x over previous
import jax
import jax.numpy as jnp
from jax import lax
from jax.experimental import pallas as pl


_CB2 = 32


def _pool_body(xa_ref, xb_ref, s_ref, o_ref):
    sel = s_ref[...]
    dn = (((2,), (0,)), ((), ()))

    def pooled(xb):
        t0 = lax.dot_general(xb, sel[0], dn, preferred_element_type=jnp.float32)
        t1 = lax.dot_general(xb, sel[1], dn, preferred_element_type=jnp.float32)
        t2 = lax.dot_general(xb, sel[2], dn, preferred_element_type=jnp.float32)
        t3 = lax.dot_general(xb, sel[3], dn, preferred_element_type=jnp.float32)
        return jnp.maximum(jnp.maximum(t0, t1), jnp.maximum(t2, t3))

    o_ref[0, :_CB2] = pooled(xa_ref[0])
    o_ref[0, _CB2:] = pooled(xb_ref[0])


def kernel(x, indices):
    b, c, s, g = x.shape
    p = indices.shape[1]
    sel = (indices.astype(jnp.int32)[:, None, :]
           == jnp.arange(g, dtype=jnp.int32)[None, :, None]).astype(x.dtype)
    return pl.pallas_call(
        _pool_body,
        grid=(b, c // (2 * _CB2)),
        in_specs=[
            pl.BlockSpec((1, _CB2, s, g), lambda i, j: (i, 2 * j, 0, 0)),
            pl.BlockSpec((1, _CB2, s, g), lambda i, j: (i, 2 * j + 1, 0, 0)),
            pl.BlockSpec((4, g, p), lambda i, j: (0, 0, 0)),
        ],
        out_specs=pl.BlockSpec((1, 2 * _CB2, s, p), lambda i, j: (i, j, 0, 0)),
        out_shape=jax.ShapeDtypeStruct((b, c, s, p), x.dtype),
    )(x, x, sel)
